# baseline (speedup 1.0000x reference)
import jax
import jax.numpy as jnp
from jax.experimental import pallas as pl
from jax.experimental.pallas import tpu as pltpu


_B = 512
_EC = 256


def _feature_kernel(x_ref, w_ref, h_ref):
    h_ref[...] = jnp.dot(
        x_ref[...], w_ref[...], preferred_element_type=jnp.float32
    ).astype(h_ref.dtype)


def _make_agg_kernel(c_half, b, ec):
    def _agg_kernel(pk, pi, pf, pc, ps, ed_ref, h_ref, b_ref, out_ref):
        del ps
        g = pl.program_id(0)
        s = pl.program_id(1)
        c = g * c_half + s

        @pl.when(pf[c] == 1)
        def _():
            out_ref[...] = jnp.broadcast_to(b_ref[...], out_ref.shape)

        @pl.when(pc[c] > 0)
        def _():
            ed = ed_ref[0]
            sl = ed[0:1, :]
            dl = ed[1:2, :]
            nv = pltpu.bitcast(ed[2:3, :], jnp.float32)
            k = pk[c]
            hk = h_ref[pl.ds(pl.multiple_of(k * b, b), b), :]
            rows = jax.lax.broadcasted_iota(jnp.int32, (b, ec), 0)
            s_t = (rows == sl).astype(jnp.bfloat16)
            g_rows = jax.lax.dot_general(
                s_t, hk,
                dimension_numbers=(((0,), (0,)), ((), ())),
                preferred_element_type=jnp.float32)
            d_n = jnp.where(rows == dl, nv, 0.0).astype(jnp.bfloat16)
            out_ref[...] += jax.lax.dot_general(
                d_n, g_rows.astype(jnp.bfloat16),
                dimension_numbers=(((1,), (0,)), ((), ())),
                preferred_element_type=jnp.float32)

    return _agg_kernel


def kernel(x, edge_index, weight, bias):
    N, nin = x.shape
    nout = weight.shape[1]
    E = edge_index.shape[1]

    nb = N // _B
    nbp = nb * nb
    half_bp = nbp // 2
    e_tot = E + N
    c_half = (e_tot + _EC - 1) // _EC + half_bp
    c_total = 2 * c_half

    src = edge_index[0].astype(jnp.int32)
    dst = edge_index[1].astype(jnp.int32)
    keep = src != dst
    loop = jnp.arange(N, dtype=jnp.int32)
    src_a = jnp.concatenate([src, loop])
    dst_a = jnp.concatenate([dst, loop])
    ew = jnp.concatenate(
        [keep.astype(jnp.float32), jnp.ones((N,), jnp.float32)])

    deg = jnp.zeros((N,), jnp.float32).at[dst_a].add(ew)
    dinv = jnp.where(deg > 0, jax.lax.rsqrt(deg), 0.0)
    norm = dinv[src_a] * ew * dinv[dst_a]

    bp = (dst_a // _B) * nb + (src_a // _B)
    bps, sl_s, dl_s, nv_s = jax.lax.sort(
        (bp, src_a % _B, dst_a % _B, norm), num_keys=1)

    starts = jnp.searchsorted(
        bps, jnp.arange(nbp + 1, dtype=jnp.int32)).astype(jnp.int32)
    cnt = jnp.diff(starts)
    nch = (cnt + _EC - 1) // _EC

    nch_h = nch.reshape(2, half_bp)
    cum_h = jnp.cumsum(nch_h, axis=1)
    c_act = cum_h[:, -1]
    s_idx = jnp.arange(c_half, dtype=jnp.int32)

    metas = []
    for h in range(2):
        cum = cum_h[h]
        bpl = jnp.minimum(
            jnp.searchsorted(cum, s_idx, side='right').astype(jnp.int32),
            half_bp - 1)
        valid = s_idx < c_act[h]
        j = s_idx - (cum[bpl] - nch_h[h][bpl])
        bp_g = h * half_bp + bpl
        start = starts[bp_g] + j * _EC
        nval = jnp.clip(cnt[bp_g] - j * _EC, 0, _EC)
        iblk = bp_g // nb
        kblk = bp_g % nb
        first = jnp.concatenate(
            [jnp.ones((1,), jnp.bool_), iblk[1:] != iblk[:-1]])
        slot = h * c_half + s_idx
        last = c_act[h] - 1
        pad = lambda a: jnp.where(valid, a, jnp.take(a, last))
        metas.append(dict(
            start=jnp.where(valid, start, 0),
            nval=jnp.where(valid, nval, 0),
            iblk=pad(iblk), kblk=pad(kblk),
            first=jnp.where(valid, first, False).astype(jnp.int32),
            slot=pad(slot)))
    meta = {k: jnp.concatenate([m[k] for m in metas]) for k in metas[0]}

    t = jnp.arange(_EC, dtype=jnp.int32)
    gpos = meta['start'][:, None] + t[None, :]
    vmask = t[None, :] < meta['nval'][:, None]
    gpos = jnp.where(vmask, gpos, 0)
    sl_c = jnp.where(vmask, jnp.take(sl_s, gpos), 0)
    dl_c = jnp.where(vmask, jnp.take(dl_s, gpos), 0)
    nv_c = jnp.where(vmask, jnp.take(nv_s, gpos), 0.0)
    edata = jnp.stack(
        [sl_c, dl_c,
         jax.lax.bitcast_convert_type(nv_c, jnp.int32)], axis=1)

    xb = x.astype(jnp.bfloat16)
    wb = weight.astype(jnp.bfloat16)
    b2 = bias.astype(jnp.float32).reshape(1, nout)

    hmat = pl.pallas_call(
        _feature_kernel,
        out_shape=jax.ShapeDtypeStruct((N, nout), jnp.bfloat16),
        grid=(N // 1024,),
        in_specs=[
            pl.BlockSpec((1024, nin), lambda i: (i, 0)),
            pl.BlockSpec((nin, nout), lambda i: (0, 0)),
        ],
        out_specs=pl.BlockSpec((1024, nout), lambda i: (i, 0)),
        compiler_params=pltpu.CompilerParams(
            dimension_semantics=("parallel",)),
    )(xb, wb)

    out = pl.pallas_call(
        _make_agg_kernel(c_half, _B, _EC),
        out_shape=jax.ShapeDtypeStruct((N, nout), jnp.float32),
        grid_spec=pltpu.PrefetchScalarGridSpec(
            num_scalar_prefetch=5,
            grid=(2, c_half),
            in_specs=[
                pl.BlockSpec((1, 3, _EC),
                             lambda g, s, pk, pi, pf, pc, ps:
                             (ps[g * c_half + s], 0, 0)),
                pl.BlockSpec((N, nout),
                             lambda g, s, pk, pi, pf, pc, ps: (0, 0)),
                pl.BlockSpec((1, nout),
                             lambda g, s, pk, pi, pf, pc, ps: (0, 0)),
            ],
            out_specs=pl.BlockSpec(
                (_B, nout),
                lambda g, s, pk, pi, pf, pc, ps: (pi[g * c_half + s], 0)),
        ),
        compiler_params=pltpu.CompilerParams(
            dimension_semantics=("parallel", "arbitrary"),
            vmem_limit_bytes=48 * 1024 * 1024),
    )(meta['kblk'], meta['iblk'], meta['first'], meta['nval'], meta['slot'],
      edata, hmat, b2)

    return out


# speedup vs baseline: 8.2138x; 8.2138x over previous
import jax
import jax.numpy as jnp
from jax.experimental import pallas as pl
from jax.experimental.pallas import tpu as pltpu


def _feature_kernel(x_ref, w_ref, h_ref):
    h_ref[...] = jnp.dot(
        x_ref[...], w_ref[...], preferred_element_type=jnp.float32
    ).astype(h_ref.dtype)


def _make_deg_kernel(n_nodes, half, nbits):
    mask_n = (1 << nbits) - 1

    def _deg_kernel(packed_ref, deg_ref):
        g = pl.program_id(0)

        def _zero(i, c):
            deg_ref[0, 0, i] = 0
            return c

        jax.lax.fori_loop(0, n_nodes, _zero, 0)

        def _body(i, c):
            v = packed_ref[g, i]
            d = v & mask_n
            w = 1 - (v >> 30)
            deg_ref[0, 0, d] = deg_ref[0, 0, d] + w
            return c

        jax.lax.fori_loop(0, half, _body, 0)

    return _deg_kernel


def _make_agg_kernel(half, nbits, p_h, p_o, chunk):
    mask_n = (1 << nbits) - 1

    def _agg_kernel(packed_ref, dinv_ref, h_ref, out_ref, acc_ref):
        g = pl.program_id(0)
        z = pl.program_id(1)

        @pl.when(z == 0)
        def _():
            acc_ref[...] = jnp.zeros(acc_ref.shape, acc_ref.dtype)

            def _body(i, c):
                v = packed_ref[g, i]
                d = v & mask_n
                s = (v >> nbits) & mask_n
                w0 = v >> 30
                n = (dinv_ref[s] * dinv_ref[d]
                     * (1 - w0).astype(jnp.float32))
                slab = h_ref[pl.ds(pl.multiple_of(s * p_h, p_h), p_h), :]
                hrow = pltpu.bitcast(slab, jnp.bfloat16).astype(jnp.float32)
                o = pl.ds(pl.multiple_of(d * p_o, p_o), p_o)
                acc_ref[o, :] = acc_ref[o, :] + hrow * n
                return c

            jax.lax.fori_loop(0, half, _body, 0)

        out_ref[0, :, :] = acc_ref[pl.ds(z * chunk, chunk), :]

    return _agg_kernel


def kernel(x, edge_index, weight, bias):
    N, nin = x.shape
    nout = weight.shape[1]
    E = edge_index.shape[1]
    nbits = (N - 1).bit_length()
    p_h = nout // 256
    p_o = nout // 128

    e_tot = E + N
    half = (e_tot + 1) // 2
    pad = 2 * half - e_tot

    src = edge_index[0].astype(jnp.int32)
    dst = edge_index[1].astype(jnp.int32)
    keep = src != dst
    loop = jnp.arange(N, dtype=jnp.int32)
    src_a = jnp.concatenate([src, loop])
    dst_a = jnp.concatenate([dst, loop])
    ew_off = jnp.concatenate(
        [jnp.where(keep, 0, 1 << 30).astype(jnp.int32),
         jnp.zeros((N,), jnp.int32)])
    packed = (src_a << nbits) | dst_a | ew_off
    if pad:
        packed = jnp.concatenate(
            [packed, jnp.full((pad,), 1 << 30, jnp.int32)])
    packed2 = packed.reshape(2, half)

    deg2 = pl.pallas_call(
        _make_deg_kernel(N, half, nbits),
        out_shape=jax.ShapeDtypeStruct((2, 1, N), jnp.int32),
        grid_spec=pltpu.PrefetchScalarGridSpec(
            num_scalar_prefetch=1,
            grid=(2, 1),
            in_specs=[],
            out_specs=pl.BlockSpec(
                (1, 1, N), lambda g, z, packed: (g, 0, 0),
                memory_space=pltpu.SMEM),
        ),
        compiler_params=pltpu.CompilerParams(
            dimension_semantics=("parallel", "arbitrary")),
    )(packed2)

    deg = (deg2[0, 0] + deg2[1, 0]).astype(jnp.float32)
    dinv = jnp.where(deg > 0, jax.lax.rsqrt(deg), 0.0)

    xb = x.astype(jnp.bfloat16)
    wb = weight.astype(jnp.bfloat16)
    bm = min(N, 1024)
    hmat = pl.pallas_call(
        _feature_kernel,
        out_shape=jax.ShapeDtypeStruct((N, nout), jnp.bfloat16),
        grid=(N // bm,),
        in_specs=[
            pl.BlockSpec((bm, nin), lambda i: (i, 0)),
            pl.BlockSpec((nin, nout), lambda i: (0, 0)),
        ],
        out_specs=pl.BlockSpec((bm, nout), lambda i: (i, 0)),
        compiler_params=pltpu.CompilerParams(
            dimension_semantics=("parallel",)),
    )(xb, wb)

    h_i32 = jax.lax.bitcast_convert_type(
        hmat.reshape(N, p_h, 2, 128).transpose(0, 1, 3, 2), jnp.int32
    ).reshape(N * p_h, 128)

    n_chunks = 16
    chunk = (N * p_o) // n_chunks
    out2 = pl.pallas_call(
        _make_agg_kernel(half, nbits, p_h, p_o, chunk),
        out_shape=jax.ShapeDtypeStruct((2, N * p_o, 128), jnp.float32),
        grid_spec=pltpu.PrefetchScalarGridSpec(
            num_scalar_prefetch=1,
            grid=(2, n_chunks),
            in_specs=[
                pl.BlockSpec(memory_space=pltpu.SMEM),
                pl.BlockSpec((N * p_h, 128), lambda g, z, packed: (0, 0)),
            ],
            out_specs=pl.BlockSpec(
                (1, chunk, 128), lambda g, z, packed: (g, z, 0)),
            scratch_shapes=[
                pltpu.VMEM((N * p_o, 128), jnp.float32)],
        ),
        compiler_params=pltpu.CompilerParams(
            dimension_semantics=("parallel", "arbitrary"),
            vmem_limit_bytes=56 * 1024 * 1024),
    )(packed2, dinv, h_i32)

    out = (out2[0] + out2[1]).reshape(N, nout) + bias[None, :].astype(
        jnp.float32)
    return out
